# baseline (speedup 1.0000x reference)
import functools

import jax
import jax.numpy as jnp
from jax import lax
from jax.experimental import pallas as pl
from jax.experimental.pallas import tpu as pltpu, tpu_sc as plsc

B, D, Q = 1024, 2048, 50000
T_K = 10.0

QP = 50048
NQV = QP // 16
PADVAL = 1e18
NW = 32
RPW = B // NW
G = 8
NG = RPW // G
DV = D // 16


def _vsqrt(x):
    i = lax.bitcast_convert_type(x, jnp.int32)
    y = lax.bitcast_convert_type(
        lax.shift_right_arithmetic(i, 1) + jnp.int32(0x1FBD1DF5), jnp.float32)
    for _ in range(3):
        y = 0.5 * (y + x / y)
    return y


_DNUMS = lax.GatherDimensionNumbers(
    offset_dims=(), collapsed_slice_dims=(0,), start_index_map=(0,))


def _permute(v, idx):
    return lax.gather(v, idx[:, None], dimension_numbers=_DNUMS,
                      slice_sizes=(1,),
                      mode=lax.GatherScatterMode.PROMISE_IN_BOUNDS)


def _allreduce(v, op):
    lanes = lax.iota(jnp.int32, 16)
    for k in (8, 4, 2, 1):
        v = op(v, _permute(v, lanes ^ k))
    return v


def _sc_kernel(feat_hbm, qm_hbm, qs_hbm, out_hbm, qmv, qsv, fbuf, obuf):
    cid = lax.axis_index("c")
    sid = lax.axis_index("s")
    wid = sid * 2 + cid
    base = wid * RPW

    pltpu.sync_copy(qm_hbm, qmv)
    pltpu.sync_copy(qs_hbm, qsv)

    zero = jnp.zeros((16,), jnp.float32)
    res = zero
    for g in range(NG):
        pltpu.sync_copy(feat_hbm.at[pl.ds(base + G * g, G), :], fbuf)

        def fsum(j, carry):
            out = []
            for r in range(G):
                v = fbuf[r, pl.ds(j * 16, 16)]
                out.append(carry[2 * r] + v)
                out.append(carry[2 * r + 1] + v * v)
            return tuple(out)

        sums = lax.fori_loop(0, DV, fsum, (zero,) * (2 * G))

        lanes = lax.iota(jnp.int32, 16)
        mb, sb = [], []
        for r in range(G):
            s1v = _allreduce(sums[2 * r], jnp.add)
            s2v = _allreduce(sums[2 * r + 1], jnp.add)
            meanv = s1v * (1.0 / D)
            varv = (s2v - s1v * meanv) * (1.0 / (D - 1))
            mb.append(meanv)
            sb.append(_vsqrt(jnp.maximum(varv, 0.0)))

        def qstep(qv, accs):
            mu = qmv[pl.ds(qv * 16, 16)]
            sg = qsv[pl.ds(qv * 16, 16)]
            out = []
            for r in range(G):
                dm = mb[r] - mu
                dsd = sb[r] - sg
                out.append(jnp.minimum(accs[r], dm * dm + dsd * dsd))
            return tuple(out)

        big = jnp.full((16,), 1e36, jnp.float32)
        accs = lax.fori_loop(0, NQV, qstep, (big,) * G)

        lane_base = (g % 2) * 8
        for r in range(G):
            t = _allreduce(jnp.exp(-T_K * _vsqrt(accs[r])), jnp.maximum)
            res = jnp.where(lanes == lane_base + r, t, res)
        if g % 2 == 1:
            obuf[pl.ds((g // 2) * 16, 16)] = res
            res = zero

    pltpu.sync_copy(obuf, out_hbm.at[pl.ds(base, RPW)])


@functools.partial(
    pl.kernel,
    out_type=jax.ShapeDtypeStruct((B,), jnp.float32),
    mesh=plsc.VectorSubcoreMesh(core_axis_name="c", subcore_axis_name="s"),
    scratch_types=[
        pltpu.VMEM((QP,), jnp.float32),
        pltpu.VMEM((QP,), jnp.float32),
        pltpu.VMEM((G, D), jnp.float32),
        pltpu.VMEM((RPW,), jnp.float32),
    ],
)
def _sc_call(feat_hbm, qm_hbm, qs_hbm, out_hbm, qmv, qsv, fbuf, obuf):
    _sc_kernel(feat_hbm, qm_hbm, qs_hbm, out_hbm, qmv, qsv, fbuf, obuf)


@jax.jit
def _run(features, queue_mus, queue_sigmas):
    qm = jnp.full((QP,), PADVAL, dtype=jnp.float32).at[:Q].set(queue_mus)
    qs = jnp.full((QP,), PADVAL, dtype=jnp.float32).at[:Q].set(queue_sigmas)
    return _sc_call(features, qm, qs)


def kernel(features, labels, pred, confidence, queue_mus, queue_sigmas):
    return _run(features, queue_mus, queue_sigmas)


# speedup vs baseline: 2.5071x; 2.5071x over previous
import functools

import jax
import jax.numpy as jnp
from jax import lax
from jax.experimental import pallas as pl
from jax.experimental.pallas import tpu as pltpu, tpu_sc as plsc

B, D, Q = 1024, 2048, 50000
T_K = 10.0
PADVAL = 1e18

QSC = 9984
QTC = Q - QSC
TC_NQT = (QTC + 127) // 128
TC_QPAD = TC_NQT * 128

NQV = QSC // 16
NW = 32
RPW = B // NW
G = 8
NG = RPW // G
DV = D // 16


def _vsqrt(x):
    i = lax.bitcast_convert_type(x, jnp.int32)
    y = lax.bitcast_convert_type(
        lax.shift_right_arithmetic(i, 1) + jnp.int32(0x1FBD1DF5), jnp.float32)
    for _ in range(3):
        y = 0.5 * (y + x / y)
    return y


_DNUMS = lax.GatherDimensionNumbers(
    offset_dims=(), collapsed_slice_dims=(0,), start_index_map=(0,))


def _permute(v, idx):
    return lax.gather(v, idx[:, None], dimension_numbers=_DNUMS,
                      slice_sizes=(1,),
                      mode=lax.GatherScatterMode.PROMISE_IN_BOUNDS)


def _allreduce(v, op):
    lanes = lax.iota(jnp.int32, 16)
    for k in (8, 4, 2, 1):
        v = op(v, _permute(v, lanes ^ k))
    return v


def _sc_kernel(feat_hbm, qm_hbm, qs_hbm, out_hbm, qmv, qsv, fbuf, obuf):
    cid = lax.axis_index("c")
    sid = lax.axis_index("s")
    wid = sid * 2 + cid
    base = wid * RPW

    pltpu.sync_copy(qm_hbm, qmv)
    pltpu.sync_copy(qs_hbm, qsv)

    zero = jnp.zeros((16,), jnp.float32)
    res = zero
    for g in range(NG):
        pltpu.sync_copy(feat_hbm.at[pl.ds(base + G * g, G), :], fbuf)

        def fsum(j, carry):
            out = []
            for r in range(G):
                v = fbuf[r, pl.ds(j * 16, 16)]
                out.append(carry[2 * r] + v)
                out.append(carry[2 * r + 1] + v * v)
            return tuple(out)

        sums = lax.fori_loop(0, DV, fsum, (zero,) * (2 * G))

        lanes = lax.iota(jnp.int32, 16)
        mb, sb = [], []
        for r in range(G):
            s1v = _allreduce(sums[2 * r], jnp.add)
            s2v = _allreduce(sums[2 * r + 1], jnp.add)
            meanv = s1v * (1.0 / D)
            varv = (s2v - s1v * meanv) * (1.0 / (D - 1))
            mb.append(meanv)
            sb.append(_vsqrt(jnp.maximum(varv, 0.0)))

        def qstep(qv, accs):
            mu = qmv[pl.ds(qv * 16, 16)]
            sg = qsv[pl.ds(qv * 16, 16)]
            out = []
            for r in range(G):
                dm = mb[r] - mu
                dsd = sb[r] - sg
                out.append(jnp.minimum(accs[r], dm * dm + dsd * dsd))
            return tuple(out)

        big = jnp.full((16,), 1e36, jnp.float32)
        accs = lax.fori_loop(0, NQV, qstep, (big,) * G)

        lane_base = (g % 2) * 8
        for r in range(G):
            t = _allreduce(jnp.exp(-T_K * _vsqrt(accs[r])), jnp.maximum)
            res = jnp.where(lanes == lane_base + r, t, res)
        if g % 2 == 1:
            obuf[pl.ds((g // 2) * 16, 16)] = res
            res = zero

    pltpu.sync_copy(obuf, out_hbm.at[pl.ds(base, RPW)])


@functools.partial(
    pl.kernel,
    out_type=jax.ShapeDtypeStruct((B,), jnp.float32),
    mesh=plsc.VectorSubcoreMesh(core_axis_name="c", subcore_axis_name="s"),
    scratch_types=[
        pltpu.VMEM((QSC,), jnp.float32),
        pltpu.VMEM((QSC,), jnp.float32),
        pltpu.VMEM((G, D), jnp.float32),
        pltpu.VMEM((RPW,), jnp.float32),
    ],
)
def _sc_call(feat_hbm, qm_hbm, qs_hbm, out_hbm, qmv, qsv, fbuf, obuf):
    _sc_kernel(feat_hbm, qm_hbm, qs_hbm, out_hbm, qmv, qsv, fbuf, obuf)


TC_ROWS = 128
TC_R = TC_ROWS // 16


def _tc_body(feat_ref, mus_ref, sigs_ref, out_ref):
    f = feat_ref[:]
    s1 = jnp.sum(f, axis=1, keepdims=True)
    s2 = jnp.sum(f * f, axis=1, keepdims=True)
    mean = s1 / D
    var = (s2 - s1 * s1 / D) / (D - 1)
    std = jnp.sqrt(var)
    mean_b = mean.astype(jnp.bfloat16)
    std_b = std.astype(jnp.bfloat16)

    mb = [jnp.broadcast_to(mean_b[16 * r:16 * r + 16, :], (16, 128)) for r in range(TC_R)]
    sb = [jnp.broadcast_to(std_b[16 * r:16 * r + 16, :], (16, 128)) for r in range(TC_R)]

    def step(qt, accs):
        mu = mus_ref[qt]
        sg = sigs_ref[qt]
        out = []
        for r in range(TC_R):
            dm = mb[r] - mu
            dsd = sb[r] - sg
            out.append(jnp.minimum(accs[r], dm * dm + dsd * dsd))
        return tuple(out)

    inf_b = jnp.asarray(3.0e38, dtype=jnp.bfloat16)
    acc0 = tuple(jnp.full((16, 128), inf_b, dtype=jnp.bfloat16) for _ in range(TC_R))
    accs = lax.fori_loop(0, TC_NQT, step, acc0)
    mind = jnp.concatenate(
        [jnp.min(a, axis=1, keepdims=True) for a in accs], axis=0)
    d2 = mind.astype(jnp.float32)
    out_ref[:] = jnp.exp(-T_K * jnp.sqrt(d2))


def _tc_search(features, qm_bf, qs_bf):
    return pl.pallas_call(
        _tc_body,
        grid=(B // TC_ROWS,),
        in_specs=[
            pl.BlockSpec((TC_ROWS, D), lambda i: (i, 0)),
            pl.BlockSpec((TC_NQT, 16, 128), lambda i: (0, 0, 0)),
            pl.BlockSpec((TC_NQT, 16, 128), lambda i: (0, 0, 0)),
        ],
        out_specs=pl.BlockSpec((TC_ROWS, 1), lambda i: (i, 0)),
        out_shape=jax.ShapeDtypeStruct((B, 1), jnp.float32),
    )(features, qm_bf, qs_bf)


def _merge_body(a_ref, b_ref, o_ref):
    o_ref[:] = jnp.maximum(a_ref[:], b_ref[:])


def _merge(a, b):
    return pl.pallas_call(
        _merge_body,
        out_shape=jax.ShapeDtypeStruct((8, 128), jnp.float32),
    )(a.reshape(8, 128), b.reshape(8, 128)).reshape(B)


@jax.jit
def _run(features, queue_mus, queue_sigmas):
    qm_sc = queue_mus[:QSC]
    qs_sc = queue_sigmas[:QSC]
    qm_tc = jnp.full((TC_QPAD,), PADVAL, jnp.float32).at[:QTC].set(queue_mus[QSC:])
    qs_tc = jnp.full((TC_QPAD,), PADVAL, jnp.float32).at[:QTC].set(queue_sigmas[QSC:])
    qm_bf = jnp.broadcast_to(qm_tc.reshape(TC_NQT, 1, 128).astype(jnp.bfloat16),
                             (TC_NQT, 16, 128))
    qs_bf = jnp.broadcast_to(qs_tc.reshape(TC_NQT, 1, 128).astype(jnp.bfloat16),
                             (TC_NQT, 16, 128))
    t_sc = _sc_call(features, qm_sc, qs_sc)
    t_tc = _tc_search(features, qm_bf, qs_bf).reshape(B)
    return _merge(t_sc, t_tc)


def kernel(features, labels, pred, confidence, queue_mus, queue_sigmas):
    return _run(features, queue_mus, queue_sigmas)
